# baseline (speedup 1.0000x reference)
import functools

import jax
import jax.numpy as jnp
from jax import lax
from jax.experimental import pallas as pl
from jax.experimental.pallas import tpu as pltpu
from jax.experimental.pallas import tpu_sc as plsc

DIM = 64
NCH = DIM // 16
BATCH = 16384
NLOOK = 4 * BATCH
NBLK = 7813
LASTB = NBLK - 1
GROWS = NLOOK + 16
WL_CAP = 8192
SEG_CAP = 1280
WIN_CAP = 512
NSEG = 8
NWIN = 62

_info = plsc.get_sparse_core_info()
_NC, _NS = _info.num_cores, _info.num_subcores
_NW = _NC * _NS


def _sc_gather(nodeT, tail, idxf):
    mesh = plsc.VectorSubcoreMesh(core_axis_name="c", subcore_axis_name="s")

    @functools.partial(
        pl.kernel,
        mesh=mesh,
        out_type=jax.ShapeDtypeStruct((GROWS, 128), jnp.float32),
        compiler_params=pltpu.CompilerParams(
            needs_layout_passes=False, use_tc_tiling_on_sc=True),
        scratch_types=[
            pltpu.VMEM((4096,), jnp.int32),
            pltpu.VMEM((WL_CAP,), jnp.int32),
            pltpu.VMEM((WL_CAP,), jnp.int32),
            pltpu.VMEM((NSEG, SEG_CAP), jnp.int32),
            pltpu.VMEM((NSEG, SEG_CAP), jnp.int32),
            pltpu.VMEM((16,), jnp.int32),
            pltpu.VMEM((WIN_CAP,), jnp.int32),
            pltpu.VMEM((WIN_CAP,), jnp.int32),
            pltpu.VMEM((2, 256, 128), jnp.float32),
            pltpu.VMEM((2, 16, 128), jnp.float32),
            pltpu.SemaphoreType.DMA,
            pltpu.SemaphoreType.DMA,
        ],
    )
    def ka(tab_hbm, tail_hbm, idx_hbm, g_hbm, piece_v, wl_n, wl_lid,
           seg_n, seg_lid, cnt_v, win_n, win_lid, blk_v, row_v, sem, osem):
        wid = lax.axis_index("s") * _NC + lax.axis_index("c")
        base_cnt = NBLK // _NW
        extra = NBLK - base_cnt * _NW
        lo = wid * base_cnt + jnp.minimum(wid, extra)
        cnt = base_cnt + jnp.where(wid < extra, 1, 0)

        iota = lax.broadcasted_iota(jnp.int32, (16,), 0)

        total = 0
        for p in range(NLOOK // 4096):
            pltpu.sync_copy(idx_hbm.at[pl.ds(p * 4096, 4096)], piece_v)

            def scan(q, off, p=p):
                v = piece_v[pl.ds(q * 16, 16)]
                rel = lax.shift_right_logical(v, 7) - lo
                msk = (rel >= 0) & (rel < cnt)
                offc = jnp.minimum(off, WL_CAP - 16)
                plsc.store_compressed(wl_n.at[pl.ds(offc, 16)], v, mask=msk)
                lid = iota + (p * 4096 + q * 16)
                plsc.store_compressed(
                    wl_lid.at[pl.ds(offc, 16)], lid, mask=msk)
                return off + plsc.all_reduce_population_count(msk)[0]

            total = lax.fori_loop(0, 4096 // 16, scan, total)
        total = jnp.minimum(total, WL_CAP)

        nvec = (total + 15) // 16
        cnts = jnp.zeros((16,), jnp.int32)
        for s in range(NSEG):
            def part(q, off, s=s):
                v = wl_n[pl.ds(q * 16, 16)]
                lid = wl_lid[pl.ds(q * 16, 16)]
                rel = lax.shift_right_logical(v, 7) - lo
                valid = (q * 16 + iota) < total
                msk = (lax.shift_right_logical(rel, 5) == s) & valid
                offc = jnp.minimum(off, SEG_CAP - 16)
                plsc.store_compressed(
                    seg_n.at[s, pl.ds(offc, 16)], v, mask=msk)
                plsc.store_compressed(
                    seg_lid.at[s, pl.ds(offc, 16)], lid, mask=msk)
                return off + plsc.all_reduce_population_count(msk)[0]

            cs = lax.fori_loop(0, nvec, part, 0)
            cnts = jnp.where(iota == s, jnp.minimum(cs, SEG_CAP), cnts)
        cnt_v[...] = cnts

        def issue_window(w):
            s2 = w & 1
            for b in range(4):
                j = jnp.minimum(lo + w * 4 + b, LASTB)

                @pl.when(j < LASTB)
                def _(j=j, s2=s2, b=b):
                    off = pl.multiple_of(j * 128, 128)
                    pltpu.async_copy(
                        tab_hbm.at[:, pl.ds(off, 128)],
                        blk_v.at[s2, pl.ds(b * 64, 64)], sem)

                @pl.when(j == LASTB)
                def _(s2=s2, b=b):
                    pltpu.async_copy(
                        tail_hbm, blk_v.at[s2, pl.ds(b * 64, 64)], sem)

        issue_window(0)
        dummy = NLOOK + iota

        def window(w, carry):
            s2 = w & 1
            for b in range(4):
                pltpu.make_async_copy(
                    tab_hbm.at[:, pl.ds(0, 128)],
                    blk_v.at[s2, pl.ds(b * 64, 64)], sem).wait()

            @pl.when(w + 1 < NWIN)
            def _():
                issue_window(w + 1)

            def group(gi, gc, s2=s2, w=w):
                gslot = gc & 1
                nv = wl_n[pl.ds(gi * 16, 16)]
                lv = wl_lid[pl.ds(gi * 16, 16)]
                relw = lax.shift_right_logical(nv, 7) - lo - w * 4
                valid = ((gi * 16 + iota) < total) & (relw >= 0) & (relw < 4)
                nvalid = plsc.all_reduce_population_count(valid)[0]

                @pl.when(nvalid > 0)
                def _():
                    lvc = jnp.where(valid, lv, NLOOK + iota)
                    nvc = jnp.where(valid, nv, (lo + w * 4) * 128)
                    rows0 = (lax.shift_right_logical(nvc, 7) - lo - w * 4) * 64
                    rows0 = jnp.where(valid, rows0, 0)
                    cols = nvc & 127
                    for d in range(DIM):
                        col = plsc.load_gather(
                            blk_v.at[s2], [rows0 + d, cols])
                        plsc.store_scatter(
                            row_v.at[gslot],
                            [iota, jnp.full((16,), d, jnp.int32)], col)
                    fence = row_v[gslot, 0, pl.ds(0, 16)]
                    row_v[gslot, 0, pl.ds(0, 16)] = fence
                    pltpu.async_copy(
                        row_v.at[gslot], g_hbm.at[lvc], osem).wait()
                return gc + 1

            carry = lax.fori_loop(0, (total + 15) // 16, group, carry)
            return carry

        lax.fori_loop(0, NWIN, window, 0)

    return ka(nodeT, tail, idxf)


def _sc_scores(g, rela, link):
    bpw = BATCH // _NW
    CH = 128
    nchunk = bpw // CH
    mesh = plsc.VectorSubcoreMesh(core_axis_name="c", subcore_axis_name="s")

    @functools.partial(
        pl.kernel,
        mesh=mesh,
        out_type=jax.ShapeDtypeStruct((BATCH,), jnp.float32),
        compiler_params=pltpu.CompilerParams(
            needs_layout_passes=False, use_tc_tiling_on_sc=False),
        scratch_types=[
            pltpu.VMEM((CH, 128), jnp.float32),
            pltpu.VMEM((CH, 128), jnp.float32),
            pltpu.VMEM((CH, 128), jnp.float32),
            pltpu.VMEM((CH, 128), jnp.float32),
            pltpu.VMEM((DIM,), jnp.float32),
            pltpu.VMEM((DIM,), jnp.float32),
            pltpu.VMEM((256,), jnp.float32),
            pltpu.VMEM((BATCH // _NW,), jnp.float32),
            pltpu.SemaphoreType.DMA,
        ],
    )
    def kb(g_hbm, rela_hbm, link_hbm, out_hbm,
           b_pi, b_pj, b_ni, b_nj, rela_v, link_v, tr_v, out_v, sem):
        wid = lax.axis_index("s") * _NC + lax.axis_index("c")
        base = wid * bpw

        pltpu.sync_copy(rela_hbm, rela_v)
        pltpu.sync_copy(link_hbm, link_v)

        iota = lax.broadcasted_iota(jnp.int32, (16,), 0)

        def allreduce(x, op):
            for step in (8, 4, 2, 1):
                tr_v[pl.ds(0, 16)] = x
                rot = plsc.load_gather(tr_v, [(iota + step) & 15])
                x = op(x, rot)
            return x

        relas = [rela_v[pl.ds(c * 16, 16)] for c in range(NCH)]
        lraw = [link_v[pl.ds(c * 16, 16)] for c in range(NCH)]
        m = lraw[0]
        for c in range(1, NCH):
            m = jnp.maximum(m, lraw[c])
        mmax = allreduce(m, jnp.maximum)
        exps = [jnp.exp(l - mmax) for l in lraw]
        tot = exps[0]
        for c in range(1, NCH):
            tot = tot + exps[c]
        denom = allreduce(tot, lax.add)
        ws = [e / denom for e in exps]

        iota16 = iota * 16
        bufs = (b_pi, b_pj, b_ni, b_nj)

        for g_ in range(nchunk):
            cps = [
                pltpu.async_copy(
                    g_hbm.at[pl.ds(t * BATCH + base + g_ * CH, CH)],
                    bufs[t], sem)
                for t in range(4)
            ]
            for cp in cps:
                cp.wait()

            def body(r16, carry, g_=g_):
                rbase = r16 * 16
                for rr in range(16):
                    row = rbase + rr
                    acc = None
                    for c in range(NCH):
                        sdim = pl.ds(c * 16, 16)
                        pi = b_pi[row, sdim]
                        pj = b_pj[row, sdim]
                        ni = b_ni[row, sdim]
                        nj = b_nj[row, sdim]
                        u = (pj + ni) - (pi + nj)
                        v = (pj + nj) - (pi + ni)
                        term = u * (v * ws[c] - relas[c])
                        acc = term if acc is None else acc + term
                    plsc.store_scatter(tr_v, [iota16 + rr], acc)
                sv = tr_v[pl.ds(0, 16)]
                for l in range(1, 16):
                    sv = sv + tr_v[pl.ds(l * 16, 16)]
                out_v[pl.ds(g_ * CH + rbase, 16)] = sv
                return carry

            lax.fori_loop(0, CH // 16, body, 0)

        pltpu.sync_copy(out_v, out_hbm.at[pl.ds(base, bpw)])

    return kb(g, rela, link)


def _softplus_tc(x2d):
    def body(x_ref, o_ref):
        x = x_ref[...]
        o_ref[...] = jnp.maximum(x, 0.0) + jnp.log1p(jnp.exp(-jnp.abs(x)))

    return pl.pallas_call(
        body, out_shape=jax.ShapeDtypeStruct(x2d.shape, jnp.float32))(x2d)


def kernel(batchVector, node_emb, rela_emb, link_emb):
    idxf = batchVector[:, 1:5].astype(jnp.int32).T.reshape(-1)
    nodeT = node_emb.T
    ntail = node_emb.shape[0] - LASTB * 128
    tail = jnp.pad(nodeT[:, LASTB * 128:], ((0, 0), (0, 128 - ntail)))
    g = _sc_gather(nodeT, tail, idxf)
    scores = _sc_scores(g, rela_emb.reshape(-1), link_emb.reshape(-1))
    loss = _softplus_tc(scores.reshape(BATCH // 128, 128)).reshape(-1)
    return loss


# speedup vs baseline: 4.0887x; 4.0887x over previous
import functools

import jax
import jax.numpy as jnp
from jax import lax
from jax.experimental import pallas as pl
from jax.experimental.pallas import tpu as pltpu
from jax.experimental.pallas import tpu_sc as plsc

DIM = 64
NCH = DIM // 16
CH = 128

_info = plsc.get_sparse_core_info()
_NC, _NS = _info.num_cores, _info.num_subcores
_NW = _NC * _NS


def _sc_scores(node_emb, idx4, rela, link):
    batch = idx4.shape[1]
    bpw = batch // _NW
    nchunk = bpw // CH
    mesh = plsc.VectorSubcoreMesh(core_axis_name="c", subcore_axis_name="s")

    @functools.partial(
        pl.kernel,
        mesh=mesh,
        out_type=jax.ShapeDtypeStruct((batch,), jnp.float32),
        compiler_params=pltpu.CompilerParams(
            needs_layout_passes=False, use_tc_tiling_on_sc=False),
        scratch_types=[
            pltpu.VMEM((4, bpw), jnp.int32),
            pltpu.VMEM((CH, DIM), jnp.float32),
            pltpu.VMEM((CH, DIM), jnp.float32),
            pltpu.VMEM((CH, DIM), jnp.float32),
            pltpu.VMEM((CH, DIM), jnp.float32),
            pltpu.VMEM((DIM,), jnp.float32),
            pltpu.VMEM((DIM,), jnp.float32),
            pltpu.VMEM((256,), jnp.float32),
            pltpu.VMEM((batch // _NW,), jnp.float32),
            pltpu.SemaphoreType.DMA,
        ],
    )
    def k(node_hbm, idx_hbm, rela_hbm, link_hbm, out_hbm,
          idx_v, b_pi, b_pj, b_ni, b_nj, rela_v, link_v, tr_v, out_v, sem):
        wid = lax.axis_index("s") * _NC + lax.axis_index("c")
        base = wid * bpw

        for t in range(4):
            pltpu.sync_copy(idx_hbm.at[t, pl.ds(base, bpw)], idx_v.at[t])
        pltpu.sync_copy(rela_hbm, rela_v)
        pltpu.sync_copy(link_hbm, link_v)

        iota = lax.broadcasted_iota(jnp.int32, (16,), 0)

        def allreduce(x, op):
            for step in (8, 4, 2, 1):
                tr_v[pl.ds(0, 16)] = x
                rot = plsc.load_gather(tr_v, [(iota + step) & 15])
                x = op(x, rot)
            return x

        relas = [rela_v[pl.ds(c * 16, 16)] for c in range(NCH)]
        lraw = [link_v[pl.ds(c * 16, 16)] for c in range(NCH)]
        m = lraw[0]
        for c in range(1, NCH):
            m = jnp.maximum(m, lraw[c])
        mmax = allreduce(m, jnp.maximum)
        exps = [jnp.exp(l - mmax) for l in lraw]
        tot = exps[0]
        for c in range(1, NCH):
            tot = tot + exps[c]
        denom = allreduce(tot, lax.add)
        ws = [e / denom for e in exps]

        iota16 = iota * 16
        bufs = (b_pi, b_pj, b_ni, b_nj)

        for g in range(nchunk):
            cps = [
                pltpu.async_copy(
                    node_hbm.at[idx_v.at[t, pl.ds(g * CH, CH)]], bufs[t], sem)
                for t in range(4)
            ]
            for cp in cps:
                cp.wait()

            def body(r16, carry, g=g):
                rbase = r16 * 16
                for rr in range(16):
                    row = rbase + rr
                    acc = None
                    for c in range(NCH):
                        s = pl.ds(c * 16, 16)
                        pi = b_pi[row, s]
                        pj = b_pj[row, s]
                        ni = b_ni[row, s]
                        nj = b_nj[row, s]
                        u = (pj + ni) - (pi + nj)
                        v = (pj + nj) - (pi + ni)
                        term = u * (v * ws[c] - relas[c])
                        acc = term if acc is None else acc + term
                    plsc.store_scatter(tr_v, [iota16 + rr], acc)
                sv = tr_v[pl.ds(0, 16)]
                for l in range(1, 16):
                    sv = sv + tr_v[pl.ds(l * 16, 16)]
                out_v[pl.ds(g * CH + rbase, 16)] = sv
                return carry

            lax.fori_loop(0, CH // 16, body, 0)

        pltpu.sync_copy(out_v, out_hbm.at[pl.ds(base, bpw)])

    return k(node_emb, idx4, rela, link)


def _softplus_tc(x2d):
    def body(x_ref, o_ref):
        x = x_ref[...]
        o_ref[...] = jnp.maximum(x, 0.0) + jnp.log1p(jnp.exp(-jnp.abs(x)))

    return pl.pallas_call(
        body, out_shape=jax.ShapeDtypeStruct(x2d.shape, jnp.float32))(x2d)


def kernel(batchVector, node_emb, rela_emb, link_emb):
    batch = batchVector.shape[0]
    idx4 = batchVector[:, 1:5].astype(jnp.int32).T
    scores = _sc_scores(
        node_emb, idx4, rela_emb.reshape(-1), link_emb.reshape(-1))
    loss = _softplus_tc(scores.reshape(batch // 128, 128)).reshape(-1)
    return loss
